# baseline (speedup 1.0000x reference)
import jax
import jax.numpy as jnp
from jax import lax
from jax.experimental import pallas as pl
from jax.experimental.pallas import tpu as pltpu
from jax.experimental.pallas import tpu_sc as plsc

N_ROWS = 100000
W = 1024
D = 512

NC, NS = 2, 16
NW = NC * NS
RPW = 800
R_SC = NW * RPW
N_TC = N_ROWS - R_SC
ROW_BLOCK = 2400
CH = 16
NCH = RPW // CH

NSLC = W // 16


def _hash_kernel(pattern_ref, proj_ref, cvec_ref, hsum_ref):
    projected = jax.lax.dot_general(
        pattern_ref[...], proj_ref[...],
        (((1,), (0,)), ((), ())),
        preferred_element_type=jnp.float32,
    )
    h = (projected > 0).astype(jnp.float32)
    cvec_ref[...] = 2.0 * h - 1.0
    hsum_ref[0, 0] = jnp.sum(h)


def _tc_scan_kernel(cvec_ref, stored_ref, val_ref, idx_ref, best_val, best_idx):
    i = pl.program_id(0)
    scores = jax.lax.dot_general(
        stored_ref[...], cvec_ref[...],
        (((1,), (1,)), ((), ())),
        preferred_element_type=jnp.float32,
    )
    m = jnp.max(scores)
    rows = jax.lax.broadcasted_iota(jnp.int32, (ROW_BLOCK, 1), 0)
    local = jnp.min(jnp.where(scores == m, rows, N_ROWS))
    gidx = i * ROW_BLOCK + local

    @pl.when((i == 0) | (m > best_val[0]))
    def _():
        best_val[0] = m
        best_idx[0] = gidx

    @pl.when(i == pl.num_programs(0) - 1)
    def _():
        val_ref[0, 0] = best_val[0]
        idx_ref[0, 0] = best_idx[0]


def _sc_chunk(buf, cvec_v, tr, gbase, bv, bi):
    accs = [jnp.zeros((16,), jnp.float32) for _ in range(CH)]
    for k in range(NSLC):
        cv = cvec_v[pl.ds(k * 16, 16)]
        for r in range(CH):
            accs[r] = accs[r] + buf[r, pl.ds(k * 16, 16)] * cv
    for r in range(CH):
        tr[r, :] = accs[r]
    lanes = lax.iota(jnp.int32, 16)
    tot = jnp.zeros((16,), jnp.float32)
    for c in range(16):
        tot = tot + plsc.load_gather(tr, [lanes, jnp.full((16,), c, jnp.int32)])
    gvec = gbase + lanes
    better = tot > bv
    bv = jnp.where(better, tot, bv)
    bi = jnp.where(better, gvec, bi)
    return bv, bi


def _sc_scan_kernel(stored_hbm, cvec_hbm, outv_hbm, outi_hbm,
                    cvec_v, tmpv, tmpi, buf_a, buf_b, tr, sem_a, sem_b):
    wid = lax.axis_index("s") * NC + lax.axis_index("c")
    base = N_TC + wid * RPW
    pltpu.sync_copy(cvec_hbm, cvec_v)
    pltpu.async_copy(stored_hbm.at[pl.ds(base, CH)], buf_a, sem_a)
    pltpu.async_copy(stored_hbm.at[pl.ds(base + CH, CH)], buf_b, sem_b)

    def pair_body(p, carry):
        bv, bi = carry
        t0 = 2 * p
        g0 = base + t0 * CH
        pltpu.make_async_copy(
            stored_hbm.at[pl.ds(g0, CH)], buf_a, sem_a).wait()
        bv, bi = _sc_chunk(buf_a, cvec_v, tr, g0, bv, bi)

        @pl.when(t0 + 2 < NCH)
        def _():
            pltpu.async_copy(
                stored_hbm.at[pl.ds(g0 + 2 * CH, CH)], buf_a, sem_a)

        g1 = g0 + CH
        pltpu.make_async_copy(
            stored_hbm.at[pl.ds(g1, CH)], buf_b, sem_b).wait()
        bv, bi = _sc_chunk(buf_b, cvec_v, tr, g1, bv, bi)

        @pl.when(t0 + 3 < NCH)
        def _():
            pltpu.async_copy(
                stored_hbm.at[pl.ds(g1 + 2 * CH, CH)], buf_b, sem_b)

        return bv, bi

    bv0 = jnp.full((16,), -jnp.inf, jnp.float32)
    bi0 = jnp.zeros((16,), jnp.int32)
    bv, bi = lax.fori_loop(0, NCH // 2, pair_body, (bv0, bi0))

    tmpv[...] = bv
    tmpi[...] = bi
    pltpu.sync_copy(tmpv, outv_hbm.at[wid])
    pltpu.sync_copy(tmpi, outi_hbm.at[wid])


def _merge_kernel(tcv_ref, tci_ref, hsum_ref, scv_ref, sci_ref, pred_any,
                  pred_ref, sim_ref, idx_ref, sem):
    sv = jnp.max(scv_ref[...])
    si = jnp.min(jnp.where(scv_ref[...] == sv, sci_ref[...], N_ROWS))
    tv = tcv_ref[0, 0]
    ti = tci_ref[0, 0]
    use_sc = sv > tv
    val = jnp.where(use_sc, sv, tv)
    idx = jnp.where(use_sc, si, ti)
    sim_ref[0, 0] = (W - hsum_ref[0, 0] + val) / W
    idx_ref[0, 0] = idx
    pltpu.make_async_copy(pred_any.at[pl.ds(idx, 1), :], pred_ref, sem).start()
    pltpu.make_async_copy(pred_any.at[pl.ds(idx, 1), :], pred_ref, sem).wait()


def kernel(pattern, hash_projections, stored_hashes, predictions):
    cvec, hsum = pl.pallas_call(
        _hash_kernel,
        out_shape=(
            jax.ShapeDtypeStruct((1, W), jnp.float32),
            jax.ShapeDtypeStruct((1, 1), jnp.float32),
        ),
        in_specs=[
            pl.BlockSpec((1, D), lambda: (0, 0)),
            pl.BlockSpec((D, W), lambda: (0, 0)),
        ],
        out_specs=(
            pl.BlockSpec((1, W), lambda: (0, 0)),
            pl.BlockSpec(memory_space=pltpu.SMEM),
        ),
    )(pattern.reshape(1, D), hash_projections)

    sc_scan = pl.kernel(
        _sc_scan_kernel,
        out_type=(
            jax.ShapeDtypeStruct((NW, 16), jnp.float32),
            jax.ShapeDtypeStruct((NW, 16), jnp.int32),
        ),
        mesh=plsc.VectorSubcoreMesh(
            core_axis_name="c", subcore_axis_name="s",
            num_cores=NC, num_subcores=NS),
        compiler_params=pltpu.CompilerParams(needs_layout_passes=False),
        scratch_types=[
            pltpu.VMEM((W,), jnp.float32),
            pltpu.VMEM((16,), jnp.float32),
            pltpu.VMEM((16,), jnp.int32),
            pltpu.VMEM((CH, W), jnp.float32),
            pltpu.VMEM((CH, W), jnp.float32),
            pltpu.VMEM((CH, 16), jnp.float32),
            pltpu.SemaphoreType.DMA,
            pltpu.SemaphoreType.DMA,
        ],
    )
    scv, sci = sc_scan(stored_hashes, cvec.reshape(W))

    tcv, tci = pl.pallas_call(
        _tc_scan_kernel,
        grid=(N_TC // ROW_BLOCK,),
        out_shape=(
            jax.ShapeDtypeStruct((1, 1), jnp.float32),
            jax.ShapeDtypeStruct((1, 1), jnp.int32),
        ),
        in_specs=[
            pl.BlockSpec((1, W), lambda i: (0, 0)),
            pl.BlockSpec((ROW_BLOCK, W), lambda i: (i, 0)),
        ],
        out_specs=(
            pl.BlockSpec(memory_space=pltpu.SMEM),
            pl.BlockSpec(memory_space=pltpu.SMEM),
        ),
        scratch_shapes=[
            pltpu.SMEM((1,), jnp.float32),
            pltpu.SMEM((1,), jnp.int32),
        ],
    )(cvec, stored_hashes)

    prediction, best_sim, best_idx = pl.pallas_call(
        _merge_kernel,
        out_shape=(
            jax.ShapeDtypeStruct((1, D), jnp.float32),
            jax.ShapeDtypeStruct((1, 1), jnp.float32),
            jax.ShapeDtypeStruct((1, 1), jnp.int32),
        ),
        in_specs=[
            pl.BlockSpec(memory_space=pltpu.SMEM),
            pl.BlockSpec(memory_space=pltpu.SMEM),
            pl.BlockSpec(memory_space=pltpu.SMEM),
            pl.BlockSpec((NW, 16), lambda: (0, 0)),
            pl.BlockSpec((NW, 16), lambda: (0, 0)),
            pl.BlockSpec(memory_space=pl.ANY),
        ],
        out_specs=(
            pl.BlockSpec((1, D), lambda: (0, 0)),
            pl.BlockSpec(memory_space=pltpu.SMEM),
            pl.BlockSpec(memory_space=pltpu.SMEM),
        ),
        scratch_shapes=[pltpu.SemaphoreType.DMA],
    )(tcv, tci, hsum, scv, sci, predictions)

    return (prediction.reshape(D), best_sim.reshape(()), best_idx.reshape(()))


# speedup vs baseline: 1.4487x; 1.4487x over previous
import jax
import jax.numpy as jnp
from jax import lax
from jax.experimental import pallas as pl
from jax.experimental.pallas import tpu as pltpu
from jax.experimental.pallas import tpu_sc as plsc

N_ROWS = 100000
W = 1024
D = 512

NC, NS = 2, 16
NW = NC * NS
RPW = 480
R_SC = NW * RPW
N_TC = N_ROWS - R_SC
ROW_BLOCK = 4232
CH = 16
NCH = RPW // CH

NSLC = W // 16


def _hash_kernel(pattern_ref, proj_ref, cvec_ref, hsum_ref):
    projected = jax.lax.dot_general(
        pattern_ref[...], proj_ref[...],
        (((1,), (0,)), ((), ())),
        preferred_element_type=jnp.float32,
    )
    h = (projected > 0).astype(jnp.float32)
    cvec_ref[...] = 2.0 * h - 1.0
    hsum_ref[0, 0] = jnp.sum(h)


def _tc_scan_kernel(cvec_ref, stored_ref, val_ref, idx_ref, best_val, best_idx):
    i = pl.program_id(0)
    scores = jax.lax.dot_general(
        stored_ref[...], cvec_ref[...],
        (((1,), (1,)), ((), ())),
        preferred_element_type=jnp.float32,
    )
    m = jnp.max(scores)
    rows = jax.lax.broadcasted_iota(jnp.int32, (ROW_BLOCK, 1), 0)
    local = jnp.min(jnp.where(scores == m, rows, N_ROWS))
    gidx = i * ROW_BLOCK + local

    @pl.when((i == 0) | (m > best_val[0]))
    def _():
        best_val[0] = m
        best_idx[0] = gidx

    @pl.when(i == pl.num_programs(0) - 1)
    def _():
        val_ref[0, 0] = best_val[0]
        idx_ref[0, 0] = best_idx[0]


def _sc_chunk(buf, cvec_v, tr, gbase, bv, bi):
    accs = [jnp.zeros((16,), jnp.float32) for _ in range(CH)]
    for k in range(NSLC):
        cv = cvec_v[pl.ds(k * 16, 16)]
        for r in range(CH):
            accs[r] = accs[r] + buf[r, pl.ds(k * 16, 16)] * cv
    for r in range(CH):
        tr[r, :] = accs[r]
    lanes = lax.iota(jnp.int32, 16)
    tot = jnp.zeros((16,), jnp.float32)
    for c in range(16):
        tot = tot + plsc.load_gather(tr, [lanes, jnp.full((16,), c, jnp.int32)])
    gvec = gbase + lanes
    better = tot > bv
    bv = jnp.where(better, tot, bv)
    bi = jnp.where(better, gvec, bi)
    return bv, bi


def _sc_scan_kernel(stored_hbm, cvec_hbm, outv_hbm, outi_hbm,
                    cvec_v, tmpv, tmpi, buf_a, buf_b, tr, sem_a, sem_b):
    wid = lax.axis_index("s") * NC + lax.axis_index("c")
    base = N_TC + wid * RPW
    pltpu.sync_copy(cvec_hbm, cvec_v)
    pltpu.async_copy(stored_hbm.at[pl.ds(base, CH)], buf_a, sem_a)
    pltpu.async_copy(stored_hbm.at[pl.ds(base + CH, CH)], buf_b, sem_b)

    def pair_body(p, carry):
        bv, bi = carry
        t0 = 2 * p
        g0 = base + t0 * CH
        pltpu.make_async_copy(
            stored_hbm.at[pl.ds(g0, CH)], buf_a, sem_a).wait()
        bv, bi = _sc_chunk(buf_a, cvec_v, tr, g0, bv, bi)

        @pl.when(t0 + 2 < NCH)
        def _():
            pltpu.async_copy(
                stored_hbm.at[pl.ds(g0 + 2 * CH, CH)], buf_a, sem_a)

        g1 = g0 + CH
        pltpu.make_async_copy(
            stored_hbm.at[pl.ds(g1, CH)], buf_b, sem_b).wait()
        bv, bi = _sc_chunk(buf_b, cvec_v, tr, g1, bv, bi)

        @pl.when(t0 + 3 < NCH)
        def _():
            pltpu.async_copy(
                stored_hbm.at[pl.ds(g1 + 2 * CH, CH)], buf_b, sem_b)

        return bv, bi

    bv0 = jnp.full((16,), -jnp.inf, jnp.float32)
    bi0 = jnp.zeros((16,), jnp.int32)
    bv, bi = lax.fori_loop(0, NCH // 2, pair_body, (bv0, bi0))

    tmpv[...] = bv
    tmpi[...] = bi
    pltpu.sync_copy(tmpv, outv_hbm.at[wid])
    pltpu.sync_copy(tmpi, outi_hbm.at[wid])


def _merge_kernel(tcv_ref, tci_ref, hsum_ref, scv_ref, sci_ref, pred_any,
                  pred_ref, sim_ref, idx_ref, sem):
    sv = jnp.max(scv_ref[...])
    si = jnp.min(jnp.where(scv_ref[...] == sv, sci_ref[...], N_ROWS))
    tv = tcv_ref[0, 0]
    ti = tci_ref[0, 0]
    use_sc = sv > tv
    val = jnp.where(use_sc, sv, tv)
    idx = jnp.where(use_sc, si, ti)
    sim_ref[0, 0] = (W - hsum_ref[0, 0] + val) / W
    idx_ref[0, 0] = idx
    pltpu.make_async_copy(pred_any.at[pl.ds(idx, 1), :], pred_ref, sem).start()
    pltpu.make_async_copy(pred_any.at[pl.ds(idx, 1), :], pred_ref, sem).wait()


def kernel(pattern, hash_projections, stored_hashes, predictions):
    cvec, hsum = pl.pallas_call(
        _hash_kernel,
        out_shape=(
            jax.ShapeDtypeStruct((1, W), jnp.float32),
            jax.ShapeDtypeStruct((1, 1), jnp.float32),
        ),
        in_specs=[
            pl.BlockSpec((1, D), lambda: (0, 0)),
            pl.BlockSpec((D, W), lambda: (0, 0)),
        ],
        out_specs=(
            pl.BlockSpec((1, W), lambda: (0, 0)),
            pl.BlockSpec(memory_space=pltpu.SMEM),
        ),
    )(pattern.reshape(1, D), hash_projections)

    sc_scan = pl.kernel(
        _sc_scan_kernel,
        out_type=(
            jax.ShapeDtypeStruct((NW, 16), jnp.float32),
            jax.ShapeDtypeStruct((NW, 16), jnp.int32),
        ),
        mesh=plsc.VectorSubcoreMesh(
            core_axis_name="c", subcore_axis_name="s",
            num_cores=NC, num_subcores=NS),
        compiler_params=pltpu.CompilerParams(needs_layout_passes=False),
        scratch_types=[
            pltpu.VMEM((W,), jnp.float32),
            pltpu.VMEM((16,), jnp.float32),
            pltpu.VMEM((16,), jnp.int32),
            pltpu.VMEM((CH, W), jnp.float32),
            pltpu.VMEM((CH, W), jnp.float32),
            pltpu.VMEM((CH, 16), jnp.float32),
            pltpu.SemaphoreType.DMA,
            pltpu.SemaphoreType.DMA,
        ],
    )
    scv, sci = sc_scan(stored_hashes, cvec.reshape(W))

    tcv, tci = pl.pallas_call(
        _tc_scan_kernel,
        grid=(N_TC // ROW_BLOCK,),
        out_shape=(
            jax.ShapeDtypeStruct((1, 1), jnp.float32),
            jax.ShapeDtypeStruct((1, 1), jnp.int32),
        ),
        in_specs=[
            pl.BlockSpec((1, W), lambda i: (0, 0)),
            pl.BlockSpec((ROW_BLOCK, W), lambda i: (i, 0)),
        ],
        out_specs=(
            pl.BlockSpec(memory_space=pltpu.SMEM),
            pl.BlockSpec(memory_space=pltpu.SMEM),
        ),
        scratch_shapes=[
            pltpu.SMEM((1,), jnp.float32),
            pltpu.SMEM((1,), jnp.int32),
        ],
    )(cvec, stored_hashes)

    prediction, best_sim, best_idx = pl.pallas_call(
        _merge_kernel,
        out_shape=(
            jax.ShapeDtypeStruct((1, D), jnp.float32),
            jax.ShapeDtypeStruct((1, 1), jnp.float32),
            jax.ShapeDtypeStruct((1, 1), jnp.int32),
        ),
        in_specs=[
            pl.BlockSpec(memory_space=pltpu.SMEM),
            pl.BlockSpec(memory_space=pltpu.SMEM),
            pl.BlockSpec(memory_space=pltpu.SMEM),
            pl.BlockSpec((NW, 16), lambda: (0, 0)),
            pl.BlockSpec((NW, 16), lambda: (0, 0)),
            pl.BlockSpec(memory_space=pl.ANY),
        ],
        out_specs=(
            pl.BlockSpec((1, D), lambda: (0, 0)),
            pl.BlockSpec(memory_space=pltpu.SMEM),
            pl.BlockSpec(memory_space=pltpu.SMEM),
        ),
        scratch_shapes=[pltpu.SemaphoreType.DMA],
    )(tcv, tci, hsum, scv, sci, predictions)

    return (prediction.reshape(D), best_sim.reshape(()), best_idx.reshape(()))


# speedup vs baseline: 1.4697x; 1.0145x over previous
import jax
import jax.numpy as jnp
from jax import lax
from jax.experimental import pallas as pl
from jax.experimental.pallas import tpu as pltpu
from jax.experimental.pallas import tpu_sc as plsc

N_ROWS = 100000
W = 1024
D = 512

NC, NS = 2, 16
NW = NC * NS
RPW = 416
R_SC = NW * RPW
N_TC = N_ROWS - R_SC
ROW_BLOCK = 4128
CH = 16
NCH = RPW // CH

NSLC = W // 16


def _hash_kernel(pattern_ref, proj_ref, cvec_ref):
    projected = jax.lax.dot_general(
        pattern_ref[...], proj_ref[...],
        (((1,), (0,)), ((), ())),
        preferred_element_type=jnp.float32,
    )
    h = (projected > 0).astype(jnp.float32)
    cvec_ref[...] = 2.0 * h - 1.0


def _tc_scan_kernel(pattern_ref, proj_ref, stored_ref,
                    val_ref, idx_ref, hsum_ref,
                    cvec, best_val, best_idx):
    i = pl.program_id(0)

    @pl.when(i == 0)
    def _():
        projected = jax.lax.dot_general(
            pattern_ref[...], proj_ref[...],
            (((1,), (0,)), ((), ())),
            preferred_element_type=jnp.float32,
        )
        h = (projected > 0).astype(jnp.float32)
        cvec[...] = 2.0 * h - 1.0
        hsum_ref[0, 0] = jnp.sum(h)

    scores = jax.lax.dot_general(
        stored_ref[...], cvec[...],
        (((1,), (1,)), ((), ())),
        preferred_element_type=jnp.float32,
    )
    m = jnp.max(scores)
    rows = jax.lax.broadcasted_iota(jnp.int32, (ROW_BLOCK, 1), 0)
    local = jnp.min(jnp.where(scores == m, rows, N_ROWS))
    gidx = i * ROW_BLOCK + local

    @pl.when((i == 0) | (m > best_val[0]))
    def _():
        best_val[0] = m
        best_idx[0] = gidx

    @pl.when(i == pl.num_programs(0) - 1)
    def _():
        val_ref[0, 0] = best_val[0]
        idx_ref[0, 0] = best_idx[0]


def _sc_chunk(buf, cvec_v, tr, gbase, bv, bi):
    accs = [jnp.zeros((16,), jnp.float32) for _ in range(CH)]
    for k in range(NSLC):
        cv = cvec_v[pl.ds(k * 16, 16)]
        for r in range(CH):
            accs[r] = accs[r] + buf[r, pl.ds(k * 16, 16)] * cv
    for r in range(CH):
        tr[r, :] = accs[r]
    lanes = lax.iota(jnp.int32, 16)
    tot = jnp.zeros((16,), jnp.float32)
    for c in range(16):
        tot = tot + plsc.load_gather(tr, [lanes, jnp.full((16,), c, jnp.int32)])
    gvec = gbase + lanes
    better = tot > bv
    bv = jnp.where(better, tot, bv)
    bi = jnp.where(better, gvec, bi)
    return bv, bi


def _sc_scan_kernel(stored_hbm, cvec_hbm, outv_hbm, outi_hbm,
                    cvec_v, tmpv, tmpi, buf_a, buf_b, tr, sem_a, sem_b):
    wid = lax.axis_index("s") * NC + lax.axis_index("c")
    base = N_TC + wid * RPW
    pltpu.sync_copy(cvec_hbm, cvec_v)
    pltpu.async_copy(stored_hbm.at[pl.ds(base, CH)], buf_a, sem_a)
    pltpu.async_copy(stored_hbm.at[pl.ds(base + CH, CH)], buf_b, sem_b)

    def pair_body(p, carry):
        bv, bi = carry
        t0 = 2 * p
        g0 = base + t0 * CH
        pltpu.make_async_copy(
            stored_hbm.at[pl.ds(g0, CH)], buf_a, sem_a).wait()
        bv, bi = _sc_chunk(buf_a, cvec_v, tr, g0, bv, bi)

        @pl.when(t0 + 2 < NCH)
        def _():
            pltpu.async_copy(
                stored_hbm.at[pl.ds(g0 + 2 * CH, CH)], buf_a, sem_a)

        g1 = g0 + CH
        pltpu.make_async_copy(
            stored_hbm.at[pl.ds(g1, CH)], buf_b, sem_b).wait()
        bv, bi = _sc_chunk(buf_b, cvec_v, tr, g1, bv, bi)

        @pl.when(t0 + 3 < NCH)
        def _():
            pltpu.async_copy(
                stored_hbm.at[pl.ds(g1 + 2 * CH, CH)], buf_b, sem_b)

        return bv, bi

    bv0 = jnp.full((16,), -jnp.inf, jnp.float32)
    bi0 = jnp.zeros((16,), jnp.int32)
    bv, bi = lax.fori_loop(0, NCH // 2, pair_body, (bv0, bi0))

    tmpv[...] = bv
    tmpi[...] = bi
    pltpu.sync_copy(tmpv, outv_hbm.at[wid])
    pltpu.sync_copy(tmpi, outi_hbm.at[wid])


def _merge_kernel(tcv_ref, tci_ref, hsum_ref, scv_ref, sci_ref, pred_any,
                  pred_ref, sim_ref, idx_ref, sem):
    sv = jnp.max(scv_ref[...])
    si = jnp.min(jnp.where(scv_ref[...] == sv, sci_ref[...], N_ROWS))
    tv = tcv_ref[0, 0]
    ti = tci_ref[0, 0]
    use_sc = sv > tv
    val = jnp.where(use_sc, sv, tv)
    idx = jnp.where(use_sc, si, ti)
    sim_ref[0, 0] = (W - hsum_ref[0, 0] + val) / W
    idx_ref[0, 0] = idx
    pltpu.make_async_copy(pred_any.at[pl.ds(idx, 1), :], pred_ref, sem).start()
    pltpu.make_async_copy(pred_any.at[pl.ds(idx, 1), :], pred_ref, sem).wait()


def kernel(pattern, hash_projections, stored_hashes, predictions):
    cvec = pl.pallas_call(
        _hash_kernel,
        out_shape=jax.ShapeDtypeStruct((1, W), jnp.float32),
        in_specs=[
            pl.BlockSpec((1, D), lambda: (0, 0)),
            pl.BlockSpec((D, W), lambda: (0, 0)),
        ],
        out_specs=pl.BlockSpec((1, W), lambda: (0, 0)),
    )(pattern.reshape(1, D), hash_projections)

    sc_scan = pl.kernel(
        _sc_scan_kernel,
        out_type=(
            jax.ShapeDtypeStruct((NW, 16), jnp.float32),
            jax.ShapeDtypeStruct((NW, 16), jnp.int32),
        ),
        mesh=plsc.VectorSubcoreMesh(
            core_axis_name="c", subcore_axis_name="s",
            num_cores=NC, num_subcores=NS),
        compiler_params=pltpu.CompilerParams(needs_layout_passes=False),
        scratch_types=[
            pltpu.VMEM((W,), jnp.float32),
            pltpu.VMEM((16,), jnp.float32),
            pltpu.VMEM((16,), jnp.int32),
            pltpu.VMEM((CH, W), jnp.float32),
            pltpu.VMEM((CH, W), jnp.float32),
            pltpu.VMEM((CH, 16), jnp.float32),
            pltpu.SemaphoreType.DMA,
            pltpu.SemaphoreType.DMA,
        ],
    )
    scv, sci = sc_scan(stored_hashes, cvec.reshape(W))

    tcv, tci, hsum = pl.pallas_call(
        _tc_scan_kernel,
        grid=(N_TC // ROW_BLOCK,),
        out_shape=(
            jax.ShapeDtypeStruct((1, 1), jnp.float32),
            jax.ShapeDtypeStruct((1, 1), jnp.int32),
            jax.ShapeDtypeStruct((1, 1), jnp.float32),
        ),
        in_specs=[
            pl.BlockSpec((1, D), lambda i: (0, 0)),
            pl.BlockSpec((D, W), lambda i: (0, 0)),
            pl.BlockSpec((ROW_BLOCK, W), lambda i: (i, 0)),
        ],
        out_specs=(
            pl.BlockSpec(memory_space=pltpu.SMEM),
            pl.BlockSpec(memory_space=pltpu.SMEM),
            pl.BlockSpec(memory_space=pltpu.SMEM),
        ),
        scratch_shapes=[
            pltpu.VMEM((1, W), jnp.float32),
            pltpu.SMEM((1,), jnp.float32),
            pltpu.SMEM((1,), jnp.int32),
        ],
    )(pattern.reshape(1, D), hash_projections, stored_hashes)

    prediction, best_sim, best_idx = pl.pallas_call(
        _merge_kernel,
        out_shape=(
            jax.ShapeDtypeStruct((1, D), jnp.float32),
            jax.ShapeDtypeStruct((1, 1), jnp.float32),
            jax.ShapeDtypeStruct((1, 1), jnp.int32),
        ),
        in_specs=[
            pl.BlockSpec(memory_space=pltpu.SMEM),
            pl.BlockSpec(memory_space=pltpu.SMEM),
            pl.BlockSpec(memory_space=pltpu.SMEM),
            pl.BlockSpec((NW, 16), lambda: (0, 0)),
            pl.BlockSpec((NW, 16), lambda: (0, 0)),
            pl.BlockSpec(memory_space=pl.ANY),
        ],
        out_specs=(
            pl.BlockSpec((1, D), lambda: (0, 0)),
            pl.BlockSpec(memory_space=pltpu.SMEM),
            pl.BlockSpec(memory_space=pltpu.SMEM),
        ),
        scratch_shapes=[pltpu.SemaphoreType.DMA],
    )(tcv, tci, hsum, scv, sci, predictions)

    return (prediction.reshape(D), best_sim.reshape(()), best_idx.reshape(()))


# speedup vs baseline: 1.5348x; 1.0443x over previous
import jax
import jax.numpy as jnp
from jax import lax
from jax.experimental import pallas as pl
from jax.experimental.pallas import tpu as pltpu
from jax.experimental.pallas import tpu_sc as plsc

N_ROWS = 100000
W = 1024
D = 512
ROW_BLOCK = 4000
NC, NS = 2, 16


def _tc_scan_kernel(pattern_ref, proj_ref, stored_ref,
                    sim_ref, idx_ref,
                    cvec, hsum, best_val, best_idx):
    i = pl.program_id(0)

    @pl.when(i == 0)
    def _():
        projected = jax.lax.dot_general(
            pattern_ref[...], proj_ref[...],
            (((1,), (0,)), ((), ())),
            preferred_element_type=jnp.float32,
        )
        h = (projected > 0).astype(jnp.float32)
        cvec[...] = 2.0 * h - 1.0
        hsum[0] = jnp.sum(h)

    scores = jax.lax.dot_general(
        stored_ref[...], cvec[...],
        (((1,), (1,)), ((), ())),
        preferred_element_type=jnp.float32,
    )
    m = jnp.max(scores)
    rows = jax.lax.broadcasted_iota(jnp.int32, (ROW_BLOCK, 1), 0)
    local = jnp.min(jnp.where(scores == m, rows, N_ROWS))
    gidx = i * ROW_BLOCK + local

    @pl.when((i == 0) | (m > best_val[0]))
    def _():
        best_val[0] = m
        best_idx[0] = gidx

    @pl.when(i == pl.num_programs(0) - 1)
    def _():
        sim_ref[0, 0] = (W - hsum[0] + best_val[0]) / W
        idx_ref[0, 0] = best_idx[0]


def _sc_gather_kernel(idx_hbm, pred_hbm, out_hbm, idx_v, row_v, sem):
    wid = lax.axis_index("s") * NC + lax.axis_index("c")

    @pl.when(wid == 0)
    def _():
        pltpu.sync_copy(idx_hbm, idx_v)
        pltpu.async_copy(pred_hbm.at[idx_v], row_v, sem).wait()
        pltpu.sync_copy(row_v, out_hbm)


def kernel(pattern, hash_projections, stored_hashes, predictions):
    best_sim, best_idx = pl.pallas_call(
        _tc_scan_kernel,
        grid=(N_ROWS // ROW_BLOCK,),
        out_shape=(
            jax.ShapeDtypeStruct((1, 1), jnp.float32),
            jax.ShapeDtypeStruct((1, 1), jnp.int32),
        ),
        in_specs=[
            pl.BlockSpec((1, D), lambda i: (0, 0)),
            pl.BlockSpec((D, W), lambda i: (0, 0)),
            pl.BlockSpec((ROW_BLOCK, W), lambda i: (i, 0)),
        ],
        out_specs=(
            pl.BlockSpec(memory_space=pltpu.SMEM),
            pl.BlockSpec(memory_space=pltpu.SMEM),
        ),
        scratch_shapes=[
            pltpu.VMEM((1, W), jnp.float32),
            pltpu.SMEM((1,), jnp.float32),
            pltpu.SMEM((1,), jnp.float32),
            pltpu.SMEM((1,), jnp.int32),
        ],
    )(pattern.reshape(1, D), hash_projections, stored_hashes)

    sc_gather = pl.kernel(
        _sc_gather_kernel,
        out_type=jax.ShapeDtypeStruct((1, D), jnp.float32),
        mesh=plsc.VectorSubcoreMesh(
            core_axis_name="c", subcore_axis_name="s",
            num_cores=NC, num_subcores=NS),
        compiler_params=pltpu.CompilerParams(needs_layout_passes=False),
        scratch_types=[
            pltpu.VMEM((1,), jnp.int32),
            pltpu.VMEM((1, D), jnp.float32),
            pltpu.SemaphoreType.DMA,
        ],
    )
    prediction = sc_gather(best_idx.reshape(1), predictions)

    return (prediction.reshape(D), best_sim.reshape(()), best_idx.reshape(()))
